# baseline (speedup 1.0000x reference)
import jax
import jax.numpy as jnp
from jax.experimental import pallas as pl

N = 10000
HID = 64
H = 4
F = HID * H
OUT = 16
AH = 128


def _gat(x, ei, Wg, al, ar, bg):
    n = x.shape[0]
    feat = (x @ Wg).reshape(n, H, HID)
    el = jnp.sum(feat * al[None], axis=-1)
    er = jnp.sum(feat * ar[None], axis=-1)
    src, dst = ei[0], ei[1]
    e = jax.nn.leaky_relu(el[src] + er[dst], negative_slope=0.2)
    m = jax.ops.segment_max(e, dst, num_segments=n)
    m = jnp.where(jnp.isfinite(m), m, 0.0)
    ex = jnp.exp(e - m[dst])
    den = jax.ops.segment_sum(ex, dst, num_segments=n)
    alpha = ex / (den[dst] + 1e-9)
    out = jax.ops.segment_sum(feat[src] * alpha[:, :, None], dst, num_segments=n)
    out = out.reshape(n, H * HID) + bg
    return jax.nn.elu(out)


def _sem(z, W1, b1, W2):
    w = jnp.tanh(z @ W1 + b1) @ W2
    w = jnp.mean(w, axis=0)
    beta = jax.nn.softmax(w, axis=0)
    return jnp.sum(beta[None] * z, axis=1)


def _final_proj_kernel(hf_ref, wp_ref, bp_ref, out_ref):
    out_ref[...] = hf_ref[...] @ wp_ref[...] + bp_ref[...][None, :]


def kernel(h, edge_index0, edge_index1, edge_index2, edge_index3, NS, W_trans, Wg0, Wg1, Wg2, Wg3, Wg4, Wg5, al0, al1, al2, al3, al4, al5, ar0, ar1, ar2, ar3, ar4, ar5, bg0, bg1, bg2, bg3, bg4, bg5, W1_s123, b1_s123, W2_s123, W1_s124, b1_s124, W2_s124, W1_sfin, b1_sfin, W2_sfin, Wp, bp):
    eis = [edge_index0, edge_index1, edge_index2, edge_index3]
    Wgs = [Wg0, Wg1, Wg2, Wg3, Wg4, Wg5]
    als = [al0, al1, al2, al3, al4, al5]
    ars = [ar0, ar1, ar2, ar3, ar4, ar5]
    bgs = [bg0, bg1, bg2, bg3, bg4, bg5]
    hp = h @ W_trans
    g123 = [eis[0], eis[1], eis[2]]
    g124 = [eis[0], eis[1], eis[3]]
    emb123 = jnp.stack([_gat(hp, g123[i], Wgs[i], als[i], ars[i], bgs[i]) for i in range(3)], axis=1)
    h123 = _sem(emb123, W1_s123, b1_s123, W2_s123)
    emb124 = jnp.stack([_gat(hp, g124[i], Wgs[3 + i], als[3 + i], ars[3 + i], bgs[3 + i]) for i in range(3)], axis=1)
    h124 = _sem(emb124, W1_s124, b1_s124, W2_s124)
    hs = jnp.stack([h123, h124], axis=1)
    hf = _sem(hs, W1_sfin, b1_sfin, W2_sfin)
    pred = pl.pallas_call(
        _final_proj_kernel,
        out_shape=jax.ShapeDtypeStruct((N, OUT), jnp.float32),
    )(hf, Wp, bp)
    return pred, hf


# speedup vs baseline: 23.6851x; 23.6851x over previous
import functools

import jax
import jax.numpy as jnp
from jax import lax
from jax.experimental import pallas as pl
from jax.experimental.pallas import tpu as pltpu
from jax.experimental.pallas import tpu_sc as plsc

N = 10000
E = 320000
IN = 128
HID = 64
H = 4
F = HID * H
OUT = 16
AH = 128

NB = 1000
NBLK = N // NB

SC_NC = 2
SC_NS = 16
NW = SC_NC * SC_NS
EPW = E // NW
CH = 80
NCHUNK = EPW // CH
NPAD = 10240
NPT = NPAD // SC_NS
ZR = 128
NPH = 12


def _hp_body(h_ref, wt_ref, o_ref):
    o_ref[...] = jnp.dot(h_ref[...], wt_ref[...], preferred_element_type=jnp.float32)


def _hp_call(h, W_trans):
    return pl.pallas_call(
        _hp_body,
        grid=(NBLK,),
        in_specs=[
            pl.BlockSpec((NB, IN), lambda nb: (nb, 0)),
            pl.BlockSpec((IN, HID), lambda nb: (0, 0)),
        ],
        out_specs=pl.BlockSpec((NB, HID), lambda nb: (nb, 0)),
        out_shape=jax.ShapeDtypeStruct((N, HID), jnp.float32),
    )(h, W_trans)


def _vw_body(wg_ref, al_ref, ar_ref, o_ref):
    wg = wg_ref[0]
    cols = []
    for h in range(H):
        blk = wg[:, h * HID:(h + 1) * HID]
        cols.append(jnp.dot(blk, al_ref[0][h][:, None], preferred_element_type=jnp.float32))
    for h in range(H):
        blk = wg[:, h * HID:(h + 1) * HID]
        cols.append(jnp.dot(blk, ar_ref[0][h][:, None], preferred_element_type=jnp.float32))
    o_ref[0] = jnp.concatenate(cols, axis=1)


def _vw_call(Wg6, al6, ar6):
    return pl.pallas_call(
        _vw_body,
        grid=(6,),
        in_specs=[
            pl.BlockSpec((1, HID, F), lambda j: (j, 0, 0)),
            pl.BlockSpec((1, H, HID), lambda j: (j, 0, 0)),
            pl.BlockSpec((1, H, HID), lambda j: (j, 0, 0)),
        ],
        out_specs=pl.BlockSpec((1, HID, 2 * H), lambda j: (j, 0, 0)),
        out_shape=jax.ShapeDtypeStruct((6, HID, 2 * H), jnp.float32),
    )(Wg6, al6, ar6)


def _feat_body(hp_ref, wg_ref, o_ref):
    o_ref[0] = jnp.dot(hp_ref[...], wg_ref[0], preferred_element_type=jnp.float32)


def _feat_call(hp, Wg6):
    return pl.pallas_call(
        _feat_body,
        grid=(2, 6, NBLK),
        in_specs=[
            pl.BlockSpec((NB, HID), lambda f, j, nb: (nb, 0)),
            pl.BlockSpec((1, HID, 128), lambda f, j, nb: (j, 0, f)),
        ],
        out_specs=pl.BlockSpec((1, NB, 128), lambda f, j, nb: (j * 2 + f, nb, 0)),
        out_shape=jax.ShapeDtypeStruct((NPH, N, 128), jnp.float32),
    )(hp, Wg6)


def _elr_body(hp_ref, vw_ref, elt_ref, ert_ref):
    lr = jnp.dot(hp_ref[...], vw_ref[0], preferred_element_type=jnp.float32)
    pad = jnp.zeros((NB, 124), jnp.float32)
    elt_ref[0] = jnp.concatenate([lr[:, 0:4], pad], axis=1)
    ert_ref[0] = jnp.concatenate([lr[:, 4:8], pad], axis=1)


def _elr_call(hp, VW):
    return pl.pallas_call(
        _elr_body,
        grid=(6, NBLK),
        in_specs=[
            pl.BlockSpec((NB, HID), lambda j, nb: (nb, 0)),
            pl.BlockSpec((1, HID, 2 * H), lambda j, nb: (j, 0, 0)),
        ],
        out_specs=[
            pl.BlockSpec((1, NB, 128), lambda j, nb: (j, nb, 0)),
            pl.BlockSpec((1, NB, 128), lambda j, nb: (j, nb, 0)),
        ],
        out_shape=[
            jax.ShapeDtypeStruct((6, N, 128), jnp.float32),
            jax.ShapeDtypeStruct((6, N, 128), jnp.float32),
        ],
    )(hp, VW)


def _sc_kernel_body(ei_ref, elt_ref, ert_ref, feat_ref, zro_ref, out1_ref,
                    out2_ref, wscr_ref, acc, src_t, dst_t, gidx_t, lgs_t,
                    lgd_t, rows_t, w128_t, wst_t, sem):
    c = lax.axis_index("c")
    s = lax.axis_index("s")
    wid = c * SC_NS + s

    z16f = jnp.zeros((16,), jnp.float32)
    iota = lax.iota(jnp.int32, 16)

    def _z2(r, _):
        for q in range(1, 8):
            w128_t[r, pl.ds(q * 16, 16)] = z16f
        return 0
    lax.fori_loop(0, CH, _z2, 0)

    def _zero_acc():
        pltpu.sync_copy(zro_ref.at[pl.ds(s * NPT, NPT)],
                        acc.at[pl.ds(s * NPT, NPT)])

    def _dump_acc(row0):
        pltpu.sync_copy(acc.at[pl.ds(s * NPT, NPT)],
                        out1_ref.at[pl.ds(row0 + s * NPT, NPT)])

    def _load_edges(j, i):
        base = j * (2 * E) + wid * EPW + i * CH
        pltpu.sync_copy(ei_ref.at[pl.ds(base, CH)], src_t)
        pltpu.sync_copy(ei_ref.at[pl.ds(base + E, CH)], dst_t)

    def _build_idx(off_s, off_d):
        def g(gg, _):
            sv = src_t[pl.ds(gg * 16, 16)]
            dv = dst_t[pl.ds(gg * 16, 16)]
            gidx_t[pl.ds(gg * 16, 16)] = sv + off_s
            gidx_t[pl.ds(CH + gg * 16, 16)] = dv + off_d
            return 0
        lax.fori_loop(0, CH // 16, g, 0)

    mask4 = jnp.where(iota < 4, 1.0, 0.0)

    def graph(j, _):
        _zero_acc()
        plsc.subcore_barrier()

        def dchunk(i, _):
            _load_edges(j, i)
            _build_idx(j * N, j * N)
            pltpu.async_copy(elt_ref.at[gidx_t.at[pl.ds(0, CH)]], lgs_t, sem).wait()
            pltpu.async_copy(ert_ref.at[gidx_t.at[pl.ds(CH, CH)]], lgd_t, sem).wait()

            def edge(ke, _2):
                rs = lgs_t[ke, pl.ds(0, 16)]
                rd = lgd_t[ke, pl.ds(0, 16)]
                sm = rs + rd
                e = jnp.where(sm > 0, sm, 0.2 * sm)
                w16 = jnp.exp(e) * mask4
                w128_t[ke, pl.ds(0, 16)] = w16
                wst_t[pl.ds(ke * 16, 16)] = w16
                return 0
            lax.fori_loop(0, CH, edge, 0)

            pltpu.sync_copy(wst_t,
                            wscr_ref.at[pl.ds((wid * EPW + i * CH) * 16, CH * 16)])
            pltpu.sync_copy(w128_t, acc.at[dst_t], add=True)
            return 0
        lax.fori_loop(0, NCHUNK, dchunk, 0)

        plsc.subcore_barrier()
        pltpu.sync_copy(acc.at[pl.ds(s * NPT, NPT)],
                        out2_ref.at[pl.ds(c * (6 * NPAD) + j * NPAD + s * NPT, NPT)])
        plsc.subcore_barrier()

        for f in range(2):
            _zero_acc()
            plsc.subcore_barrier()

            def hchunk(i, _):
                _load_edges(j, i)
                _build_idx((2 * j + f) * N, 0)
                pltpu.sync_copy(
                    wscr_ref.at[pl.ds((wid * EPW + i * CH) * 16, CH * 16)], wst_t)
                pltpu.async_copy(feat_ref.at[gidx_t.at[pl.ds(0, CH)]], rows_t,
                                 sem).wait()

                def edge(ke, _2):
                    wv = wst_t[pl.ds(ke * 16, 16)]
                    b0 = jnp.broadcast_to(wv[2 * f], (16,))
                    b1 = jnp.broadcast_to(wv[2 * f + 1], (16,))
                    for q in range(8):
                        v = rows_t[ke, pl.ds(q * 16, 16)]
                        rows_t[ke, pl.ds(q * 16, 16)] = v * (b0 if q < 4 else b1)
                    return 0
                lax.fori_loop(0, CH, edge, 0)

                pltpu.sync_copy(rows_t, acc.at[dst_t], add=True)
                return 0
            lax.fori_loop(0, NCHUNK, hchunk, 0)

            plsc.subcore_barrier()
            _dump_acc(c * (NPH * NPAD) + (2 * j + f) * NPAD)
            plsc.subcore_barrier()
        return 0

    lax.fori_loop(0, 6, graph, 0)


def _sc_call(ei6f, eltf, ertf, feat2d, zro):
    mesh = plsc.VectorSubcoreMesh(
        core_axis_name="c", subcore_axis_name="s", num_cores=SC_NC, num_subcores=SC_NS)
    kfn = functools.partial(
        pl.kernel,
        out_type=(
            jax.ShapeDtypeStruct((SC_NC * NPH * NPAD, 128), jnp.float32),
            jax.ShapeDtypeStruct((SC_NC * 6 * NPAD, 128), jnp.float32),
            jax.ShapeDtypeStruct((E * 16,), jnp.float32),
        ),
        mesh=mesh,
        scratch_types=[
            pltpu.VMEM_SHARED((NPAD, 128), jnp.float32),
            pltpu.VMEM((CH,), jnp.int32),
            pltpu.VMEM((CH,), jnp.int32),
            pltpu.VMEM((2 * CH,), jnp.int32),
            pltpu.VMEM((CH, 128), jnp.float32),
            pltpu.VMEM((CH, 128), jnp.float32),
            pltpu.VMEM((CH, 128), jnp.float32),
            pltpu.VMEM((CH, 128), jnp.float32),
            pltpu.VMEM((CH * 16,), jnp.float32),
            pltpu.SemaphoreType.DMA,
        ],
    )(_sc_kernel_body)
    return kfn(ei6f, eltf, ertf, feat2d, zro)


def _emb_body(o1_ref, o2_ref, bg_ref, w1_ref, b1_ref, w2_ref, emb_ref, ss_ref):
    nb = pl.program_id(1)
    u0 = o1_ref[0, 0] + o1_ref[1, 0]
    u1 = o1_ref[0, 1] + o1_ref[1, 1]
    den = o2_ref[0, 0, :, 0:8] + o2_ref[1, 0, :, 0:8]
    pieces = []
    for h in range(H):
        u = u0 if h < 2 else u1
        col = u[:, (h % 2) * HID:(h % 2) * HID + HID]
        pieces.append(col / (den[:, h:h + 1] + 1e-9))
    x = jnp.concatenate(pieces, axis=1) + bg_ref[0, 0][None, :]
    emb = jnp.where(x > 0, x, jnp.exp(x) - 1.0)
    emb_ref[0] = emb
    t = jnp.tanh(jnp.dot(emb, w1_ref[0], preferred_element_type=jnp.float32)
                 + b1_ref[0, 0][None, :])
    sv = jnp.dot(t, w2_ref[0], preferred_element_type=jnp.float32)
    s2 = jnp.sum(sv).reshape(1, 1, 1)

    @pl.when(nb == 0)
    def _():
        ss_ref[...] = jnp.zeros((1, 1, 1), jnp.float32)
    ss_ref[...] += s2


def _emb_call(out1, out2, bg6, W1s, b1s, W2s):
    return pl.pallas_call(
        _emb_body,
        grid=(6, NBLK),
        in_specs=[
            pl.BlockSpec((2, 2, NB, 128), lambda j, nb: (0, j, nb, 0)),
            pl.BlockSpec((2, 1, NB, 128), lambda j, nb: (0, j, nb, 0)),
            pl.BlockSpec((1, 1, F), lambda j, nb: (j, 0, 0)),
            pl.BlockSpec((1, F, AH), lambda j, nb: (j // 3, 0, 0)),
            pl.BlockSpec((1, 1, AH), lambda j, nb: (j // 3, 0, 0)),
            pl.BlockSpec((1, AH, 1), lambda j, nb: (j // 3, 0, 0)),
        ],
        out_specs=[
            pl.BlockSpec((1, NB, F), lambda j, nb: (j, nb, 0)),
            pl.BlockSpec((1, 1, 1), lambda j, nb: (j, 0, 0)),
        ],
        out_shape=[
            jax.ShapeDtypeStruct((6, N, F), jnp.float32),
            jax.ShapeDtypeStruct((6, 1, 1), jnp.float32),
        ],
    )(out1, out2, bg6, W1s, b1s, W2s)


def _softmax3(ss):
    m = jnp.max(ss)
    b = jnp.exp(ss - m)
    return b / jnp.sum(b)


def _comb_body(emb_ref, ss_ref, w1_ref, b1_ref, w2_ref, h123_ref, h124_ref, sf_ref):
    nb = pl.program_id(0)
    b123 = _softmax3(ss_ref[0:3, 0, :] / N)
    b124 = _softmax3(ss_ref[3:6, 0, :] / N)
    e = emb_ref[...]
    h123 = e[0] * b123[0:1] + e[1] * b123[1:2] + e[2] * b123[2:3]
    h124 = e[3] * b124[0:1] + e[4] * b124[1:2] + e[5] * b124[2:3]
    h123_ref[...] = h123
    h124_ref[...] = h124
    t1 = jnp.tanh(jnp.dot(h123, w1_ref[...], preferred_element_type=jnp.float32)
                  + b1_ref[...][None, :])
    s1 = jnp.sum(jnp.dot(t1, w2_ref[...], preferred_element_type=jnp.float32))
    t2 = jnp.tanh(jnp.dot(h124, w1_ref[...], preferred_element_type=jnp.float32)
                  + b1_ref[...][None, :])
    s2 = jnp.sum(jnp.dot(t2, w2_ref[...], preferred_element_type=jnp.float32))
    sv = jnp.concatenate([s1.reshape(1, 1, 1), s2.reshape(1, 1, 1)], axis=0)

    @pl.when(nb == 0)
    def _():
        sf_ref[...] = jnp.zeros((2, 1, 1), jnp.float32)
    sf_ref[...] += sv


def _comb_call(emb, ssum, W1f, b1f, W2f):
    return pl.pallas_call(
        _comb_body,
        grid=(NBLK,),
        in_specs=[
            pl.BlockSpec((6, NB, F), lambda nb: (0, nb, 0)),
            pl.BlockSpec((6, 1, 1), lambda nb: (0, 0, 0)),
            pl.BlockSpec((F, AH), lambda nb: (0, 0)),
            pl.BlockSpec((AH,), lambda nb: (0,)),
            pl.BlockSpec((AH, 1), lambda nb: (0, 0)),
        ],
        out_specs=[
            pl.BlockSpec((NB, F), lambda nb: (nb, 0)),
            pl.BlockSpec((NB, F), lambda nb: (nb, 0)),
            pl.BlockSpec((2, 1, 1), lambda nb: (0, 0, 0)),
        ],
        out_shape=[
            jax.ShapeDtypeStruct((N, F), jnp.float32),
            jax.ShapeDtypeStruct((N, F), jnp.float32),
            jax.ShapeDtypeStruct((2, 1, 1), jnp.float32),
        ],
    )(emb, ssum, W1f, b1f, W2f)


def _fin_body(h123_ref, h124_ref, sf_ref, wp_ref, bp_ref, pred_ref, hf_ref):
    ss = sf_ref[...] / N
    m = jnp.max(ss)
    b = jnp.exp(ss - m)
    b = b / jnp.sum(b)
    hf = h123_ref[...] * b[0] + h124_ref[...] * b[1]
    hf_ref[...] = hf
    pred_ref[...] = (jnp.dot(hf, wp_ref[...], preferred_element_type=jnp.float32)
                     + bp_ref[...][None, :])


def _fin_call(h123, h124, sf, Wp, bp):
    return pl.pallas_call(
        _fin_body,
        grid=(NBLK,),
        in_specs=[
            pl.BlockSpec((NB, F), lambda nb: (nb, 0)),
            pl.BlockSpec((NB, F), lambda nb: (nb, 0)),
            pl.BlockSpec((2, 1, 1), lambda nb: (0, 0, 0)),
            pl.BlockSpec((F, OUT), lambda nb: (0, 0)),
            pl.BlockSpec((OUT,), lambda nb: (0,)),
        ],
        out_specs=[
            pl.BlockSpec((NB, OUT), lambda nb: (nb, 0)),
            pl.BlockSpec((NB, F), lambda nb: (nb, 0)),
        ],
        out_shape=[
            jax.ShapeDtypeStruct((N, OUT), jnp.float32),
            jax.ShapeDtypeStruct((N, F), jnp.float32),
        ],
    )(h123, h124, sf, Wp, bp)


def kernel(h, edge_index0, edge_index1, edge_index2, edge_index3, NS, W_trans,
           Wg0, Wg1, Wg2, Wg3, Wg4, Wg5,
           al0, al1, al2, al3, al4, al5,
           ar0, ar1, ar2, ar3, ar4, ar5,
           bg0, bg1, bg2, bg3, bg4, bg5,
           W1_s123, b1_s123, W2_s123,
           W1_s124, b1_s124, W2_s124,
           W1_sfin, b1_sfin, W2_sfin,
           Wp, bp):
    del NS
    Wg6 = jnp.stack([Wg0, Wg1, Wg2, Wg3, Wg4, Wg5])
    al6 = jnp.stack([al0, al1, al2, al3, al4, al5])
    ar6 = jnp.stack([ar0, ar1, ar2, ar3, ar4, ar5])
    bg6 = jnp.stack([bg0, bg1, bg2, bg3, bg4, bg5])[:, None, :]
    ei6f = jnp.stack([edge_index0, edge_index1, edge_index2,
                      edge_index0, edge_index1, edge_index3]).reshape(-1)
    W1s = jnp.stack([W1_s123, W1_s124])
    b1s = jnp.stack([b1_s123, b1_s124])[:, None, :]
    W2s = jnp.stack([W2_s123, W2_s124])

    hp = _hp_call(h, W_trans)
    VW = _vw_call(Wg6, al6, ar6)
    feat = _feat_call(hp, Wg6)
    ELT, ERT = _elr_call(hp, VW)

    feat2d = feat.reshape(NPH * N, 128)
    eltf = ELT.reshape(6 * N, 128)
    ertf = ERT.reshape(6 * N, 128)

    zro = jnp.zeros((NPAD, 128), jnp.float32)
    out1, out2, _wscr = _sc_call(ei6f, eltf, ertf, feat2d, zro)
    out1 = out1.reshape(SC_NC, NPH, NPAD, 128)
    out2 = out2.reshape(SC_NC, 6, NPAD, 128)

    emb, ssum = _emb_call(out1, out2, bg6, W1s, b1s, W2s)
    h123, h124, sf = _comb_call(emb, ssum, W1_sfin, b1_sfin, W2_sfin)
    pred, hf = _fin_call(h123, h124, sf, Wp, bp)
    return pred, hf
